# baseline (speedup 1.0000x reference)
import functools

import jax
import jax.numpy as jnp
from jax import lax
from jax.experimental import pallas as pl
from jax.experimental.pallas import tpu as pltpu
from jax.experimental.pallas import tpu_sc as plsc

N = 100000
D = 128
SEG = 1024
NC = 2
NS = 16
NW = NC * NS
CHUNK = 80
NCHUNK = N // CHUNK
KMAX = -(-NCHUNK // NW)
SEG_PER_TILE = SEG // NS

_MESH = plsc.VectorSubcoreMesh(core_axis_name="c", subcore_axis_name="s")


@functools.partial(
    pl.kernel,
    mesh=_MESH,
    out_type=(
        jax.ShapeDtypeStruct((NC, SEG, D), jnp.float32),
        jax.ShapeDtypeStruct((N,), jnp.int32),
    ),
    scratch_types=[
        pltpu.VMEM((CHUNK, D), jnp.float32),
        pltpu.VMEM((CHUNK,), jnp.int32),
        pltpu.VMEM((CHUNK,), jnp.int32),
        pltpu.VMEM((CHUNK,), jnp.int32),
        pltpu.VMEM((16,), jnp.int32),
        pltpu.VMEM_SHARED((SEG, D), jnp.float32),
    ],
)
def _sc_scatter(x_hbm, act_hbm, batch_hbm, nl_hbm, zeros_hbm,
                partials_hbm, idx_hbm,
                xbuf, abuf, bbuf, ibuf, nlbuf, acc):
    cid = lax.axis_index("c")
    sid = lax.axis_index("s")
    w = sid * NC + cid
    pltpu.sync_copy(zeros_hbm.at[pl.ds(sid * SEG_PER_TILE, SEG_PER_TILE)],
                    acc.at[pl.ds(sid * SEG_PER_TILE, SEG_PER_TILE)])
    pltpu.sync_copy(nl_hbm.at[pl.ds(0, 16)], nlbuf)
    plsc.subcore_barrier()
    nl0 = nlbuf[pl.ds(0, 16)][0]

    def chunk_body(k, carry):
        c = w + NW * k

        @pl.when(c < NCHUNK)
        def _():
            base = c * CHUNK
            pltpu.sync_copy(x_hbm.at[pl.ds(base, CHUNK)], xbuf)
            pltpu.sync_copy(act_hbm.at[pl.ds(base, CHUNK)], abuf)
            pltpu.sync_copy(batch_hbm.at[pl.ds(base, CHUNK)], bbuf)
            for j in range(CHUNK // 16):
                av = abuf[pl.ds(j * 16, 16)]
                bv = bbuf[pl.ds(j * 16, 16)]
                ibuf[pl.ds(j * 16, 16)] = av + bv * nl0
            pltpu.sync_copy(ibuf, idx_hbm.at[pl.ds(base, CHUNK)])
            pltpu.sync_copy(xbuf, acc.at[ibuf], add=True)

        return carry

    lax.fori_loop(0, KMAX, chunk_body, 0)
    plsc.subcore_barrier()
    pltpu.sync_copy(acc.at[pl.ds(sid * SEG_PER_TILE, SEG_PER_TILE)],
                    partials_hbm.at[cid, pl.ds(sid * SEG_PER_TILE, SEG_PER_TILE)])


@functools.partial(
    pl.kernel,
    mesh=_MESH,
    out_type=jax.ShapeDtypeStruct((N, D), jnp.float32),
    scratch_types=[
        pltpu.VMEM((CHUNK,), jnp.int32),
        pltpu.VMEM((CHUNK, D), jnp.float32),
        pltpu.SemaphoreType.DMA,
    ],
)
def _sc_gather(idx_hbm, table_hbm, out_hbm, ibuf, rbuf, sem):
    cid = lax.axis_index("c")
    sid = lax.axis_index("s")
    w = sid * NC + cid

    def chunk_body(k, carry):
        c = w + NW * k

        @pl.when(c < NCHUNK)
        def _():
            base = c * CHUNK
            pltpu.sync_copy(idx_hbm.at[pl.ds(base, CHUNK)], ibuf)
            pltpu.async_copy(table_hbm.at[ibuf], rbuf, sem).wait()
            pltpu.sync_copy(rbuf, out_hbm.at[pl.ds(base, CHUNK)])

        return carry

    lax.fori_loop(0, KMAX, chunk_body, 0)


def _mlp(x, w1, b1, w2, b2):
    h = lax.dot_general(x, w1, (((1,), (1,)), ((), ())),
                        preferred_element_type=jnp.float32,
                        precision=lax.Precision.HIGHEST)
    h = jnp.maximum(h + b1, 0.0)
    o = lax.dot_general(h, w2, (((1,), (1,)), ((), ())),
                        preferred_element_type=jnp.float32,
                        precision=lax.Precision.HIGHEST)
    return o + b2


def _rho_body(p_ref, w1_ref, b1_ref, w2_ref, b2_ref, o_ref):
    p = p_ref[0] + p_ref[1]
    o_ref[:] = _mlp(p, w1_ref[:], b1_ref[:], w2_ref[:], b2_ref[:])


def _phi_body(x_ref, rb_ref, w1_ref, b1_ref, w2_ref, b2_ref, o_ref):
    o_ref[:] = _mlp(x_ref[:], w1_ref[:], b1_ref[:], w2_ref[:], b2_ref[:]) + rb_ref[:]


_BLK = 1000


def kernel(x, activation_idx, batch, num_layers,
           W1p, b1p, W2p, b2p, W1r, b1r, W2r, b2r):
    zeros = jnp.zeros((SEG, D), jnp.float32)
    partials, idx = _sc_scatter(x, activation_idx, batch, num_layers, zeros)

    rho_table = pl.pallas_call(
        _rho_body,
        out_shape=jax.ShapeDtypeStruct((SEG, D), jnp.float32),
    )(partials, W1r, b1r.reshape(1, D), W2r, b2r.reshape(1, D))

    rho_brod = _sc_gather(idx, rho_table)

    wspec = pl.BlockSpec((D, D), lambda i: (0, 0))
    bspec = pl.BlockSpec((1, D), lambda i: (0, 0))
    out = pl.pallas_call(
        _phi_body,
        grid=(N // _BLK,),
        in_specs=[
            pl.BlockSpec((_BLK, D), lambda i: (i, 0)),
            pl.BlockSpec((_BLK, D), lambda i: (i, 0)),
            wspec, bspec, wspec, bspec,
        ],
        out_specs=pl.BlockSpec((_BLK, D), lambda i: (i, 0)),
        out_shape=jax.ShapeDtypeStruct((N, D), jnp.float32),
    )(x, rho_brod, W1p, b1p.reshape(1, D), W2p, b2p.reshape(1, D))
    return out


# speedup vs baseline: 1.4731x; 1.4731x over previous
import functools

import jax
import jax.numpy as jnp
from jax import lax
from jax.experimental import pallas as pl
from jax.experimental.pallas import tpu as pltpu
from jax.experimental.pallas import tpu_sc as plsc

N = 100000
D = 128
SEG = 1024
NC = 2
NS = 16
NW = NC * NS
CHUNK = 80
NCHUNK = N // CHUNK
KMAX = -(-NCHUNK // NW)
SEG_PER_TILE = SEG // NS

_MESH = plsc.VectorSubcoreMesh(core_axis_name="c", subcore_axis_name="s")


@functools.partial(
    pl.kernel,
    mesh=_MESH,
    out_type=(
        jax.ShapeDtypeStruct((NC, SEG, D), jnp.float32),
        jax.ShapeDtypeStruct((N,), jnp.int32),
    ),
    scratch_types=[
        pltpu.VMEM((CHUNK, D), jnp.float32),
        pltpu.VMEM((CHUNK,), jnp.int32),
        pltpu.VMEM((CHUNK,), jnp.int32),
        pltpu.VMEM((CHUNK,), jnp.int32),
        pltpu.VMEM((16,), jnp.int32),
        pltpu.VMEM_SHARED((SEG, D), jnp.float32),
    ],
)
def _sc_scatter(x_hbm, act_hbm, batch_hbm, nl_hbm, zeros_hbm,
                partials_hbm, idx_hbm,
                xbuf, abuf, bbuf, ibuf, nlbuf, acc):
    cid = lax.axis_index("c")
    sid = lax.axis_index("s")
    w = sid * NC + cid
    pltpu.sync_copy(zeros_hbm.at[pl.ds(sid * SEG_PER_TILE, SEG_PER_TILE)],
                    acc.at[pl.ds(sid * SEG_PER_TILE, SEG_PER_TILE)])
    pltpu.sync_copy(nl_hbm.at[pl.ds(0, 16)], nlbuf)
    plsc.subcore_barrier()
    nl0 = nlbuf[pl.ds(0, 16)][0]

    def chunk_body(k, carry):
        c = w + NW * k

        @pl.when(c < NCHUNK)
        def _():
            base = c * CHUNK
            pltpu.sync_copy(x_hbm.at[pl.ds(base, CHUNK)], xbuf)
            pltpu.sync_copy(act_hbm.at[pl.ds(base, CHUNK)], abuf)
            pltpu.sync_copy(batch_hbm.at[pl.ds(base, CHUNK)], bbuf)
            for j in range(CHUNK // 16):
                av = abuf[pl.ds(j * 16, 16)]
                bv = bbuf[pl.ds(j * 16, 16)]
                ibuf[pl.ds(j * 16, 16)] = av + bv * nl0
            pltpu.sync_copy(ibuf, idx_hbm.at[pl.ds(base, CHUNK)])
            pltpu.sync_copy(xbuf, acc.at[ibuf], add=True)

        return carry

    lax.fori_loop(0, KMAX, chunk_body, 0)
    plsc.subcore_barrier()
    pltpu.sync_copy(acc.at[pl.ds(sid * SEG_PER_TILE, SEG_PER_TILE)],
                    partials_hbm.at[cid, pl.ds(sid * SEG_PER_TILE, SEG_PER_TILE)])


@functools.partial(
    pl.kernel,
    mesh=_MESH,
    out_type=jax.ShapeDtypeStruct((N, D), jnp.float32),
    scratch_types=[
        pltpu.VMEM((CHUNK,), jnp.int32),
        pltpu.VMEM((CHUNK, D), jnp.float32),
        pltpu.SemaphoreType.DMA,
    ],
)
def _sc_gather(idx_hbm, table_hbm, out_hbm, ibuf, rbuf, sem):
    cid = lax.axis_index("c")
    sid = lax.axis_index("s")
    w = sid * NC + cid

    def chunk_body(k, carry):
        c = w + NW * k

        @pl.when(c < NCHUNK)
        def _():
            base = c * CHUNK
            pltpu.sync_copy(idx_hbm.at[pl.ds(base, CHUNK)], ibuf)
            pltpu.async_copy(table_hbm.at[ibuf], rbuf, sem).wait()
            pltpu.sync_copy(rbuf, out_hbm.at[pl.ds(base, CHUNK)])

        return carry

    lax.fori_loop(0, KMAX, chunk_body, 0)


def _mlp(x, w1, b1, w2, b2):
    h = lax.dot_general(x, w1, (((1,), (1,)), ((), ())),
                        preferred_element_type=jnp.float32)
    h = jnp.maximum(h + b1, 0.0)
    o = lax.dot_general(h, w2, (((1,), (1,)), ((), ())),
                        preferred_element_type=jnp.float32)
    return o + b2


def _rho_body(p_ref, w1_ref, b1_ref, w2_ref, b2_ref, o_ref):
    p = p_ref[0] + p_ref[1]
    o_ref[:] = _mlp(p, w1_ref[:], b1_ref[:], w2_ref[:], b2_ref[:])


def _phi_body(x_ref, rb_ref, w1_ref, b1_ref, w2_ref, b2_ref, o_ref):
    o_ref[:] = _mlp(x_ref[:], w1_ref[:], b1_ref[:], w2_ref[:], b2_ref[:]) + rb_ref[:]


_BLK = 1000


def kernel(x, activation_idx, batch, num_layers,
           W1p, b1p, W2p, b2p, W1r, b1r, W2r, b2r):
    zeros = jnp.zeros((SEG, D), jnp.float32)
    partials, idx = _sc_scatter(x, activation_idx, batch, num_layers, zeros)

    rho_table = pl.pallas_call(
        _rho_body,
        out_shape=jax.ShapeDtypeStruct((SEG, D), jnp.float32),
    )(partials, W1r, b1r.reshape(1, D), W2r, b2r.reshape(1, D))

    rho_brod = _sc_gather(idx, rho_table)

    wspec = pl.BlockSpec((D, D), lambda i: (0, 0))
    bspec = pl.BlockSpec((1, D), lambda i: (0, 0))
    out = pl.pallas_call(
        _phi_body,
        grid=(N // _BLK,),
        in_specs=[
            pl.BlockSpec((_BLK, D), lambda i: (i, 0)),
            pl.BlockSpec((_BLK, D), lambda i: (i, 0)),
            wspec, bspec, wspec, bspec,
        ],
        out_specs=pl.BlockSpec((_BLK, D), lambda i: (i, 0)),
        out_shape=jax.ShapeDtypeStruct((N, D), jnp.float32),
    )(x, rho_brod, W1p, b1p.reshape(1, D), W2p, b2p.reshape(1, D))
    return out


# speedup vs baseline: 1.9740x; 1.3400x over previous
import functools

import jax
import jax.numpy as jnp
from jax import lax
from jax.experimental import pallas as pl
from jax.experimental.pallas import tpu as pltpu
from jax.experimental.pallas import tpu_sc as plsc

N = 100000
D = 128
SEG = 1024
NC = 2
NS = 16
NW = NC * NS
S = 160
G = S // 80
NSUP = N // S
NFULL = NSUP % NW
KHI = -(-NSUP // NW)
SEG_PER_TILE = SEG // NS

_MESH = plsc.VectorSubcoreMesh(core_axis_name="c", subcore_axis_name="s")


def _compute_idx(abuf, bbuf, ibuf, nl0):
    for j in range(G):
        for t in range(5):
            av = abuf[pl.ds(j * 80 + t * 16, 16)]
            bv = bbuf[pl.ds(j * 80 + t * 16, 16)]
            ibuf[j, pl.ds(t * 16, 16)] = av + bv * nl0


@functools.partial(
    pl.kernel,
    mesh=_MESH,
    out_type=jax.ShapeDtypeStruct((NC, SEG, D), jnp.float32),
    scratch_types=[
        pltpu.VMEM((S, D), jnp.float32),
        pltpu.VMEM((S, D), jnp.float32),
        pltpu.VMEM((S,), jnp.int32),
        pltpu.VMEM((S,), jnp.int32),
        pltpu.VMEM((S,), jnp.int32),
        pltpu.VMEM((S,), jnp.int32),
        pltpu.VMEM((G, 80), jnp.int32),
        pltpu.VMEM((G, 80), jnp.int32),
        pltpu.VMEM((16,), jnp.int32),
        pltpu.VMEM_SHARED((SEG, D), jnp.float32),
        pltpu.SemaphoreType.DMA,
        pltpu.SemaphoreType.DMA,
        pltpu.SemaphoreType.DMA,
    ],
)
def _sc_scatter(x_hbm, act_hbm, batch_hbm, nl_hbm, zeros_hbm, partials_hbm,
                xbuf0, xbuf1, abuf0, abuf1, bbuf0, bbuf1, ibuf0, ibuf1,
                nlbuf, acc, isem0, isem1, ssem):
    cid = lax.axis_index("c")
    sid = lax.axis_index("s")
    w = sid * NC + cid
    nsup = jnp.where(w < NFULL, KHI, KHI - 1)
    xbufs, abufs, bbufs, ibufs = (xbuf0, xbuf1), (abuf0, abuf1), (bbuf0, bbuf1), (ibuf0, ibuf1)
    isems = (isem0, isem1)

    def stage(s_id, b):
        base = (w + NW * s_id) * S
        pltpu.async_copy(x_hbm.at[pl.ds(base, S)], xbufs[b], isems[b])
        pltpu.async_copy(act_hbm.at[pl.ds(base, S)], abufs[b], isems[b])
        pltpu.async_copy(batch_hbm.at[pl.ds(base, S)], bbufs[b], isems[b])

    pltpu.sync_copy(zeros_hbm.at[pl.ds(sid * SEG_PER_TILE, SEG_PER_TILE)],
                    acc.at[pl.ds(sid * SEG_PER_TILE, SEG_PER_TILE)])
    pltpu.sync_copy(nl_hbm.at[pl.ds(0, 16)], nlbuf)
    for b in range(2):
        stage(b, b)
    nl0 = nlbuf[pl.ds(0, 16)][0]
    plsc.subcore_barrier()

    def round_body(g, carry):
        for b in range(2):
            s_id = 2 * g + b

            @pl.when(s_id < nsup)
            def _():
                pltpu.make_async_copy(x_hbm.at[pl.ds(0, S)], xbufs[b], isems[b]).wait()
                pltpu.make_async_copy(act_hbm.at[pl.ds(0, S)], abufs[b], isems[b]).wait()
                pltpu.make_async_copy(batch_hbm.at[pl.ds(0, S)], bbufs[b], isems[b]).wait()
                _compute_idx(abufs[b], bbufs[b], ibufs[b], nl0)
                for j in range(G):
                    pltpu.async_copy(xbufs[b].at[pl.ds(j * 80, 80)],
                                     acc.at[ibufs[b].at[j]], ssem, add=True)
                for j in range(G):
                    pltpu.make_async_copy(xbufs[b].at[pl.ds(j * 80, 80)],
                                          acc.at[ibufs[b].at[j]], ssem).wait()

                @pl.when(s_id + 2 < nsup)
                def _():
                    stage(s_id + 2, b)

        return carry

    lax.fori_loop(0, KHI // 2, round_body, 0)
    plsc.subcore_barrier()
    pltpu.sync_copy(acc.at[pl.ds(sid * SEG_PER_TILE, SEG_PER_TILE)],
                    partials_hbm.at[cid, pl.ds(sid * SEG_PER_TILE, SEG_PER_TILE)])


@functools.partial(
    pl.kernel,
    mesh=_MESH,
    out_type=jax.ShapeDtypeStruct((N, D), jnp.float32),
    scratch_types=[
        pltpu.VMEM((S, D), jnp.float32),
        pltpu.VMEM((S, D), jnp.float32),
        pltpu.VMEM((S,), jnp.int32),
        pltpu.VMEM((S,), jnp.int32),
        pltpu.VMEM((S,), jnp.int32),
        pltpu.VMEM((S,), jnp.int32),
        pltpu.VMEM((G, 80), jnp.int32),
        pltpu.VMEM((G, 80), jnp.int32),
        pltpu.VMEM((16,), jnp.int32),
        pltpu.SemaphoreType.DMA,
        pltpu.SemaphoreType.DMA,
        pltpu.SemaphoreType.DMA,
        pltpu.SemaphoreType.DMA,
        pltpu.SemaphoreType.DMA,
    ],
)
def _sc_gather(act_hbm, batch_hbm, nl_hbm, table_hbm, out_hbm,
               rbuf0, rbuf1, abuf0, abuf1, bbuf0, bbuf1, ibuf0, ibuf1,
               nlbuf, isem0, isem1, gsem, osem0, osem1):
    cid = lax.axis_index("c")
    sid = lax.axis_index("s")
    w = sid * NC + cid
    nsup = jnp.where(w < NFULL, KHI, KHI - 1)
    rbufs, abufs, bbufs, ibufs = (rbuf0, rbuf1), (abuf0, abuf1), (bbuf0, bbuf1), (ibuf0, ibuf1)
    isems, osems = (isem0, isem1), (osem0, osem1)

    def stage(s_id, b):
        base = (w + NW * s_id) * S
        pltpu.async_copy(act_hbm.at[pl.ds(base, S)], abufs[b], isems[b])
        pltpu.async_copy(batch_hbm.at[pl.ds(base, S)], bbufs[b], isems[b])

    pltpu.sync_copy(nl_hbm.at[pl.ds(0, 16)], nlbuf)
    for b in range(2):
        stage(b, b)
    nl0 = nlbuf[pl.ds(0, 16)][0]

    def round_body(g, carry):
        for b in range(2):
            s_id = 2 * g + b

            @pl.when(s_id < nsup)
            def _():
                base = (w + NW * s_id) * S
                pltpu.make_async_copy(act_hbm.at[pl.ds(0, S)], abufs[b], isems[b]).wait()
                pltpu.make_async_copy(batch_hbm.at[pl.ds(0, S)], bbufs[b], isems[b]).wait()
                _compute_idx(abufs[b], bbufs[b], ibufs[b], nl0)

                @pl.when(s_id >= 2)
                def _():
                    pltpu.make_async_copy(rbufs[b], out_hbm.at[pl.ds(0, S)], osems[b]).wait()

                for j in range(G):
                    pltpu.async_copy(table_hbm.at[ibufs[b].at[j]],
                                     rbufs[b].at[pl.ds(j * 80, 80)], gsem)
                for j in range(G):
                    pltpu.make_async_copy(table_hbm.at[ibufs[b].at[j]],
                                          rbufs[b].at[pl.ds(j * 80, 80)], gsem).wait()
                pltpu.async_copy(rbufs[b], out_hbm.at[pl.ds(base, S)], osems[b])

                @pl.when(s_id + 2 < nsup)
                def _():
                    stage(s_id + 2, b)

        return carry

    lax.fori_loop(0, KHI // 2, round_body, 0)
    for b in range(2):
        pltpu.make_async_copy(rbufs[b], out_hbm.at[pl.ds(0, S)], osems[b]).wait()


def _mlp(x, w1, b1, w2, b2):
    h = lax.dot_general(x, w1, (((1,), (1,)), ((), ())),
                        preferred_element_type=jnp.float32)
    h = jnp.maximum(h + b1, 0.0)
    o = lax.dot_general(h, w2, (((1,), (1,)), ((), ())),
                        preferred_element_type=jnp.float32)
    return o + b2


def _rho_body(p_ref, w1_ref, b1_ref, w2_ref, b2_ref, o_ref):
    p = p_ref[0] + p_ref[1]
    o_ref[:] = _mlp(p, w1_ref[:], b1_ref[:], w2_ref[:], b2_ref[:])


def _phi_body(x_ref, rb_ref, w1_ref, b1_ref, w2_ref, b2_ref, o_ref):
    o_ref[:] = _mlp(x_ref[:], w1_ref[:], b1_ref[:], w2_ref[:], b2_ref[:]) + rb_ref[:]


_BLK = 1000


def kernel(x, activation_idx, batch, num_layers,
           W1p, b1p, W2p, b2p, W1r, b1r, W2r, b2r):
    zeros = jnp.zeros((SEG, D), jnp.float32)
    partials = _sc_scatter(x, activation_idx, batch, num_layers, zeros)

    rho_table = pl.pallas_call(
        _rho_body,
        out_shape=jax.ShapeDtypeStruct((SEG, D), jnp.float32),
    )(partials, W1r, b1r.reshape(1, D), W2r, b2r.reshape(1, D))

    rho_brod = _sc_gather(activation_idx, batch, num_layers, rho_table)

    wspec = pl.BlockSpec((D, D), lambda i: (0, 0))
    bspec = pl.BlockSpec((1, D), lambda i: (0, 0))
    out = pl.pallas_call(
        _phi_body,
        grid=(N // _BLK,),
        in_specs=[
            pl.BlockSpec((_BLK, D), lambda i: (i, 0)),
            pl.BlockSpec((_BLK, D), lambda i: (i, 0)),
            wspec, bspec, wspec, bspec,
        ],
        out_specs=pl.BlockSpec((_BLK, D), lambda i: (i, 0)),
        out_shape=jax.ShapeDtypeStruct((N, D), jnp.float32),
    )(x, rho_brod, W1p, b1p.reshape(1, D), W2p, b2p.reshape(1, D))
    return out


# speedup vs baseline: 4.1714x; 2.1132x over previous
import functools

import jax
import jax.numpy as jnp
from jax import lax
from jax.experimental import pallas as pl
from jax.experimental.pallas import tpu as pltpu
from jax.experimental.pallas import tpu_sc as plsc

N = 100000
D = 128
SEG = 1024
SEGU = 512
NC = 2
NS = 16
NW = NC * NS
S = 160
G = S // 80
NSUP = N // S
NFULL = NSUP % NW
KHI = -(-NSUP // NW)
SEGU_PER_TILE = SEGU // NS

_MESH = plsc.VectorSubcoreMesh(core_axis_name="c", subcore_axis_name="s")


def _compute_idx(abuf, bbuf, ibuf, nl0):
    for j in range(G):
        for t in range(5):
            av = abuf[pl.ds(j * 80 + t * 16, 16)]
            bv = bbuf[pl.ds(j * 80 + t * 16, 16)]
            ibuf[j, pl.ds(t * 16, 16)] = av + bv * nl0


@functools.partial(
    pl.kernel,
    mesh=_MESH,
    out_type=(
        jax.ShapeDtypeStruct((NC, SEGU, D), jnp.float32),
        jax.ShapeDtypeStruct((N,), jnp.int32),
    ),
    scratch_types=[
        pltpu.VMEM((S, D), jnp.float32),
        pltpu.VMEM((S, D), jnp.float32),
        pltpu.VMEM((S,), jnp.int32),
        pltpu.VMEM((S,), jnp.int32),
        pltpu.VMEM((S,), jnp.int32),
        pltpu.VMEM((S,), jnp.int32),
        pltpu.VMEM((G, 80), jnp.int32),
        pltpu.VMEM((G, 80), jnp.int32),
        pltpu.VMEM((16,), jnp.int32),
        pltpu.VMEM_SHARED((SEGU, D), jnp.float32),
        pltpu.SemaphoreType.DMA,
        pltpu.SemaphoreType.DMA,
        pltpu.SemaphoreType.DMA,
        pltpu.SemaphoreType.DMA,
        pltpu.SemaphoreType.DMA,
    ],
)
def _sc_scatter(x_hbm, act_hbm, batch_hbm, nl_hbm, zeros_hbm,
                partials_hbm, idx_hbm,
                xbuf0, xbuf1, abuf0, abuf1, bbuf0, bbuf1, ibuf0, ibuf1,
                nlbuf, acc, isem0, isem1, ssem, osem0, osem1):
    cid = lax.axis_index("c")
    sid = lax.axis_index("s")
    w = sid * NC + cid
    nsup = jnp.where(w < NFULL, KHI, KHI - 1)
    xbufs, abufs, bbufs, ibufs = (xbuf0, xbuf1), (abuf0, abuf1), (bbuf0, bbuf1), (ibuf0, ibuf1)
    isems, osems = (isem0, isem1), (osem0, osem1)

    def stage(s_id, b):
        base = (w + NW * s_id) * S
        pltpu.async_copy(x_hbm.at[pl.ds(base, S)], xbufs[b], isems[b])
        pltpu.async_copy(act_hbm.at[pl.ds(base, S)], abufs[b], isems[b])
        pltpu.async_copy(batch_hbm.at[pl.ds(base, S)], bbufs[b], isems[b])

    pltpu.sync_copy(zeros_hbm.at[pl.ds(sid * SEGU_PER_TILE, SEGU_PER_TILE)],
                    acc.at[pl.ds(sid * SEGU_PER_TILE, SEGU_PER_TILE)])
    pltpu.sync_copy(nl_hbm.at[pl.ds(0, 16)], nlbuf)
    for b in range(2):
        stage(b, b)
    nl0 = nlbuf[pl.ds(0, 16)][0]
    plsc.subcore_barrier()

    def round_body(g, carry):
        for b in range(2):
            s_id = 2 * g + b

            @pl.when(s_id < nsup)
            def _():
                base = (w + NW * s_id) * S
                pltpu.make_async_copy(x_hbm.at[pl.ds(0, S)], xbufs[b], isems[b]).wait()
                pltpu.make_async_copy(act_hbm.at[pl.ds(0, S)], abufs[b], isems[b]).wait()
                pltpu.make_async_copy(batch_hbm.at[pl.ds(0, S)], bbufs[b], isems[b]).wait()

                @pl.when(s_id >= 2)
                def _():
                    for j in range(G):
                        pltpu.make_async_copy(ibufs[b].at[j], idx_hbm.at[pl.ds(0, 80)],
                                              osems[b]).wait()

                _compute_idx(abufs[b], bbufs[b], ibufs[b], nl0)
                for j in range(G):
                    pltpu.async_copy(ibufs[b].at[j],
                                     idx_hbm.at[pl.ds(base + j * 80, 80)], osems[b])
                for j in range(G):
                    pltpu.async_copy(xbufs[b].at[pl.ds(j * 80, 80)],
                                     acc.at[ibufs[b].at[j]], ssem, add=True)
                for j in range(G):
                    pltpu.make_async_copy(xbufs[b].at[pl.ds(j * 80, 80)],
                                          acc.at[ibufs[b].at[j]], ssem).wait()

                @pl.when(s_id + 2 < nsup)
                def _():
                    stage(s_id + 2, b)

        return carry

    lax.fori_loop(0, KHI // 2, round_body, 0)
    for b in range(2):
        for j in range(G):
            pltpu.make_async_copy(ibufs[b].at[j], idx_hbm.at[pl.ds(0, 80)],
                                  osems[b]).wait()
    plsc.subcore_barrier()
    pltpu.sync_copy(acc.at[pl.ds(sid * SEGU_PER_TILE, SEGU_PER_TILE)],
                    partials_hbm.at[cid, pl.ds(sid * SEGU_PER_TILE, SEGU_PER_TILE)])


def _mlp(x, w1, b1, w2, b2):
    h = lax.dot_general(x, w1, (((1,), (1,)), ((), ())),
                        preferred_element_type=jnp.float32)
    h = jnp.maximum(h + b1, 0.0)
    o = lax.dot_general(h, w2, (((1,), (1,)), ((), ())),
                        preferred_element_type=jnp.float32)
    return o + b2


def _rho_body(p_ref, w1_ref, b1_ref, w2_ref, b2_ref, o_ref):
    p = p_ref[0] + p_ref[1]
    o_ref[:] = _mlp(p, w1_ref[:], b1_ref[:], w2_ref[:], b2_ref[:]).astype(jnp.bfloat16)


def _phi_body(x_ref, idx_ref, tab_ref, w1_ref, b1_ref, w2_ref, b2_ref, o_ref):
    phi = _mlp(x_ref[:], w1_ref[:], b1_ref[:], w2_ref[:], b2_ref[:])
    idxv = idx_ref[0, 0, :]
    onehot = (idxv[:, None] ==
              lax.broadcasted_iota(jnp.int32, (1, SEGU), 1)).astype(jnp.bfloat16)
    rho = lax.dot_general(onehot, tab_ref[:], (((1,), (0,)), ((), ())),
                          preferred_element_type=jnp.float32)
    o_ref[:] = phi + rho


_BLK = 1000


def kernel(x, activation_idx, batch, num_layers,
           W1p, b1p, W2p, b2p, W1r, b1r, W2r, b2r):
    zeros = jnp.zeros((SEGU, D), jnp.float32)
    partials, idx = _sc_scatter(x, activation_idx, batch, num_layers, zeros)

    rho_table = pl.pallas_call(
        _rho_body,
        out_shape=jax.ShapeDtypeStruct((SEGU, D), jnp.bfloat16),
    )(partials, W1r, b1r.reshape(1, D), W2r, b2r.reshape(1, D))

    idx3 = idx.reshape(N // _BLK, 1, _BLK)
    wspec = pl.BlockSpec((D, D), lambda i: (0, 0))
    bspec = pl.BlockSpec((1, D), lambda i: (0, 0))
    out = pl.pallas_call(
        _phi_body,
        grid=(N // _BLK,),
        in_specs=[
            pl.BlockSpec((_BLK, D), lambda i: (i, 0)),
            pl.BlockSpec((1, 1, _BLK), lambda i: (i, 0, 0)),
            pl.BlockSpec((SEGU, D), lambda i: (0, 0)),
            wspec, bspec, wspec, bspec,
        ],
        out_specs=pl.BlockSpec((_BLK, D), lambda i: (i, 0)),
        out_shape=jax.ShapeDtypeStruct((N, D), jnp.float32),
    )(x, idx3, rho_table, W1p, b1p.reshape(1, D), W2p, b2p.reshape(1, D))
    return out


# speedup vs baseline: 5.1504x; 1.2347x over previous
import functools

import jax
import jax.numpy as jnp
from jax import lax
from jax.experimental import pallas as pl
from jax.experimental.pallas import tpu as pltpu
from jax.experimental.pallas import tpu_sc as plsc

N = 100000
D = 128
SEG = 1024
SEGU = 512
NC = 2
NS = 16
NW = NC * NS
S = 160
G = S // 80
NSUP = N // S
NFULL = NSUP % NW
KHI = -(-NSUP // NW)
SEGU_PER_TILE = SEGU // NS

_MESH = plsc.VectorSubcoreMesh(core_axis_name="c", subcore_axis_name="s")


def _compute_idx(abuf, bbuf, ibuf, nl0):
    for j in range(G):
        for t in range(5):
            av = abuf[pl.ds(j * 80 + t * 16, 16)]
            bv = bbuf[pl.ds(j * 80 + t * 16, 16)]
            ibuf[j, pl.ds(t * 16, 16)] = av + bv * nl0


@functools.partial(
    pl.kernel,
    mesh=_MESH,
    out_type=(
        jax.ShapeDtypeStruct((NC, SEGU, D), jnp.float32),
        jax.ShapeDtypeStruct((N,), jnp.int32),
    ),
    scratch_types=[
        pltpu.VMEM((S, D), jnp.float32),
        pltpu.VMEM((S, D), jnp.float32),
        pltpu.VMEM((S,), jnp.int32),
        pltpu.VMEM((S,), jnp.int32),
        pltpu.VMEM((S,), jnp.int32),
        pltpu.VMEM((S,), jnp.int32),
        pltpu.VMEM((G, 80), jnp.int32),
        pltpu.VMEM((G, 80), jnp.int32),
        pltpu.VMEM((16,), jnp.int32),
        pltpu.VMEM_SHARED((SEGU, D), jnp.float32),
        pltpu.SemaphoreType.DMA,
        pltpu.SemaphoreType.DMA,
        pltpu.SemaphoreType.DMA,
        pltpu.SemaphoreType.DMA,
        pltpu.SemaphoreType.DMA,
    ],
)
def _sc_scatter(x_hbm, act_hbm, batch_hbm, nl_hbm, zeros_hbm,
                partials_hbm, idx_hbm,
                xbuf0, xbuf1, abuf0, abuf1, bbuf0, bbuf1, ibuf0, ibuf1,
                nlbuf, acc, isem0, isem1, ssem, osem0, osem1):
    cid = lax.axis_index("c")
    sid = lax.axis_index("s")
    w = sid * NC + cid
    nsup = jnp.where(w < NFULL, KHI, KHI - 1)
    xbufs, abufs, bbufs, ibufs = (xbuf0, xbuf1), (abuf0, abuf1), (bbuf0, bbuf1), (ibuf0, ibuf1)
    isems, osems = (isem0, isem1), (osem0, osem1)

    def stage(s_id, b):
        base = (w + NW * s_id) * S
        pltpu.async_copy(x_hbm.at[pl.ds(base, S)], xbufs[b], isems[b])
        pltpu.async_copy(act_hbm.at[pl.ds(base, S)], abufs[b], isems[b])
        pltpu.async_copy(batch_hbm.at[pl.ds(base, S)], bbufs[b], isems[b])

    pltpu.sync_copy(zeros_hbm.at[pl.ds(sid * SEGU_PER_TILE, SEGU_PER_TILE)],
                    acc.at[pl.ds(sid * SEGU_PER_TILE, SEGU_PER_TILE)])
    pltpu.sync_copy(nl_hbm.at[pl.ds(0, 16)], nlbuf)
    for b in range(2):
        stage(b, b)
    nl0 = nlbuf[pl.ds(0, 16)][0]
    plsc.subcore_barrier()

    def round_body(g, carry):
        for b in range(2):
            s_id = 2 * g + b

            @pl.when(s_id < nsup)
            def _():
                base = (w + NW * s_id) * S
                pltpu.make_async_copy(x_hbm.at[pl.ds(0, S)], xbufs[b], isems[b]).wait()
                pltpu.make_async_copy(act_hbm.at[pl.ds(0, S)], abufs[b], isems[b]).wait()
                pltpu.make_async_copy(batch_hbm.at[pl.ds(0, S)], bbufs[b], isems[b]).wait()

                @pl.when(s_id >= 2)
                def _():
                    for j in range(G):
                        pltpu.make_async_copy(ibufs[b].at[j], idx_hbm.at[pl.ds(0, 80)],
                                              osems[b]).wait()

                _compute_idx(abufs[b], bbufs[b], ibufs[b], nl0)
                for j in range(G):
                    pltpu.async_copy(ibufs[b].at[j],
                                     idx_hbm.at[pl.ds(base + j * 80, 80)], osems[b])
                for j in range(G):
                    pltpu.async_copy(xbufs[b].at[pl.ds(j * 80, 80)],
                                     acc.at[ibufs[b].at[j]], ssem, add=True)
                for j in range(G):
                    pltpu.make_async_copy(xbufs[b].at[pl.ds(j * 80, 80)],
                                          acc.at[ibufs[b].at[j]], ssem).wait()

                @pl.when(s_id + 2 < nsup)
                def _():
                    stage(s_id + 2, b)

        return carry

    lax.fori_loop(0, KHI // 2, round_body, 0)
    for b in range(2):
        for j in range(G):
            pltpu.make_async_copy(ibufs[b].at[j], idx_hbm.at[pl.ds(0, 80)],
                                  osems[b]).wait()
    plsc.subcore_barrier()
    pltpu.sync_copy(acc.at[pl.ds(sid * SEGU_PER_TILE, SEGU_PER_TILE)],
                    partials_hbm.at[cid, pl.ds(sid * SEGU_PER_TILE, SEGU_PER_TILE)])


def _mlp_f32(x, w1, b1, w2, b2):
    h = lax.dot_general(x, w1, (((1,), (1,)), ((), ())),
                        preferred_element_type=jnp.float32)
    h = jnp.maximum(h + b1, 0.0)
    o = lax.dot_general(h, w2, (((1,), (1,)), ((), ())),
                        preferred_element_type=jnp.float32)
    return o + b2


def _mlp_bf16(x, w1, b1, w2, b2):
    h = lax.dot_general(x.astype(jnp.bfloat16), w1, (((1,), (1,)), ((), ())),
                        preferred_element_type=jnp.float32)
    h = jnp.maximum(h + b1, 0.0)
    o = lax.dot_general(h.astype(jnp.bfloat16), w2, (((1,), (1,)), ((), ())),
                        preferred_element_type=jnp.float32)
    return o + b2


def _rho_body(p_ref, w1_ref, b1_ref, w2_ref, b2_ref, o_ref):
    p = p_ref[0] + p_ref[1]
    o_ref[:] = _mlp_f32(p, w1_ref[:], b1_ref[:],
                        w2_ref[:], b2_ref[:]).astype(jnp.bfloat16)


def _phi_body(x_ref, idx_ref, tab_ref, w1p_ref, b1p_ref, w2p_ref, b2p_ref, o_ref):
    phi = _mlp_bf16(x_ref[:], w1p_ref[:], b1p_ref[:], w2p_ref[:], b2p_ref[:])
    idxv = idx_ref[0, 0, :]
    onehot = (idxv[:, None] ==
              lax.broadcasted_iota(jnp.int32, (1, SEGU), 1)).astype(jnp.bfloat16)
    rho = lax.dot_general(onehot, tab_ref[:], (((1,), (0,)), ((), ())),
                          preferred_element_type=jnp.float32)
    o_ref[:] = phi + rho


_BLK = 2000


def kernel(x, activation_idx, batch, num_layers,
           W1p, b1p, W2p, b2p, W1r, b1r, W2r, b2r):
    zeros = jnp.zeros((SEGU, D), jnp.float32)
    partials, idx = _sc_scatter(x, activation_idx, batch, num_layers, zeros)

    rho_table = pl.pallas_call(
        _rho_body,
        out_shape=jax.ShapeDtypeStruct((SEGU, D), jnp.bfloat16),
    )(partials, W1r, b1r.reshape(1, D), W2r, b2r.reshape(1, D))

    idx3 = idx.reshape(N // _BLK, 1, _BLK)
    wspec = pl.BlockSpec((D, D), lambda i: (0, 0))
    bspec = pl.BlockSpec((1, D), lambda i: (0, 0))
    out = pl.pallas_call(
        _phi_body,
        grid=(N // _BLK,),
        in_specs=[
            pl.BlockSpec((_BLK, D), lambda i: (i, 0)),
            pl.BlockSpec((1, 1, _BLK), lambda i: (i, 0, 0)),
            pl.BlockSpec((SEGU, D), lambda i: (0, 0)),
            wspec, bspec, wspec, bspec,
        ],
        out_specs=pl.BlockSpec((_BLK, D), lambda i: (i, 0)),
        out_shape=jax.ShapeDtypeStruct((N, D), jnp.float32),
    )(x, idx3, rho_table,
      W1p.astype(jnp.bfloat16), b1p.reshape(1, D),
      W2p.astype(jnp.bfloat16), b2p.reshape(1, D))
    return out


# speedup vs baseline: 5.2009x; 1.0098x over previous
import functools

import jax
import jax.numpy as jnp
from jax import lax
from jax.experimental import pallas as pl
from jax.experimental.pallas import tpu as pltpu
from jax.experimental.pallas import tpu_sc as plsc

N = 100000
D = 128
SEG = 1024
SEGU = 512
NC = 2
NS = 16
NW = NC * NS
S = 400
G = S // 80
NSUP = N // S
NFULL = NSUP % NW
KHI = -(-NSUP // NW)
SEGU_PER_TILE = SEGU // NS

_MESH = plsc.VectorSubcoreMesh(core_axis_name="c", subcore_axis_name="s")


def _compute_idx(abuf, bbuf, ibuf, nl0):
    for j in range(G):
        for t in range(5):
            av = abuf[pl.ds(j * 80 + t * 16, 16)]
            bv = bbuf[pl.ds(j * 80 + t * 16, 16)]
            ibuf[j, pl.ds(t * 16, 16)] = av + bv * nl0


@functools.partial(
    pl.kernel,
    mesh=_MESH,
    out_type=jax.ShapeDtypeStruct((NC, SEGU, D), jnp.float32),
    scratch_types=[
        pltpu.VMEM((S, D), jnp.float32),
        pltpu.VMEM((S, D), jnp.float32),
        pltpu.VMEM((S,), jnp.int32),
        pltpu.VMEM((S,), jnp.int32),
        pltpu.VMEM((S,), jnp.int32),
        pltpu.VMEM((S,), jnp.int32),
        pltpu.VMEM((G, 80), jnp.int32),
        pltpu.VMEM((G, 80), jnp.int32),
        pltpu.VMEM((16,), jnp.int32),
        pltpu.VMEM_SHARED((SEGU, D), jnp.float32),
        pltpu.SemaphoreType.DMA,
        pltpu.SemaphoreType.DMA,
        pltpu.SemaphoreType.DMA,
    ],
)
def _sc_scatter(x_hbm, act_hbm, batch_hbm, nl_hbm, zeros_hbm,
                partials_hbm,
                xbuf0, xbuf1, abuf0, abuf1, bbuf0, bbuf1, ibuf0, ibuf1,
                nlbuf, acc, isem0, isem1, ssem):
    cid = lax.axis_index("c")
    sid = lax.axis_index("s")
    w = sid * NC + cid
    nsup = jnp.where(w < NFULL, KHI, KHI - 1)
    xbufs, abufs, bbufs, ibufs = (xbuf0, xbuf1), (abuf0, abuf1), (bbuf0, bbuf1), (ibuf0, ibuf1)
    isems = (isem0, isem1)

    def stage(s_id, b):
        base = (w + NW * s_id) * S
        pltpu.async_copy(x_hbm.at[pl.ds(base, S)], xbufs[b], isems[b])
        pltpu.async_copy(act_hbm.at[pl.ds(base, S)], abufs[b], isems[b])
        pltpu.async_copy(batch_hbm.at[pl.ds(base, S)], bbufs[b], isems[b])

    pltpu.sync_copy(zeros_hbm.at[pl.ds(sid * SEGU_PER_TILE, SEGU_PER_TILE)],
                    acc.at[pl.ds(sid * SEGU_PER_TILE, SEGU_PER_TILE)])
    pltpu.sync_copy(nl_hbm.at[pl.ds(0, 16)], nlbuf)
    for b in range(2):
        stage(b, b)
    nl0 = nlbuf[pl.ds(0, 16)][0]
    plsc.subcore_barrier()

    def round_body(g, carry):
        for b in range(2):
            s_id = 2 * g + b

            @pl.when(s_id < nsup)
            def _():
                pltpu.make_async_copy(x_hbm.at[pl.ds(0, S)], xbufs[b], isems[b]).wait()
                pltpu.make_async_copy(act_hbm.at[pl.ds(0, S)], abufs[b], isems[b]).wait()
                pltpu.make_async_copy(batch_hbm.at[pl.ds(0, S)], bbufs[b], isems[b]).wait()

                _compute_idx(abufs[b], bbufs[b], ibufs[b], nl0)
                for j in range(G):
                    pltpu.async_copy(xbufs[b].at[pl.ds(j * 80, 80)],
                                     acc.at[ibufs[b].at[j]], ssem, add=True)
                for j in range(G):
                    pltpu.make_async_copy(xbufs[b].at[pl.ds(j * 80, 80)],
                                          acc.at[ibufs[b].at[j]], ssem).wait()

                @pl.when(s_id + 2 < nsup)
                def _():
                    stage(s_id + 2, b)

        return carry

    lax.fori_loop(0, KHI // 2, round_body, 0)
    plsc.subcore_barrier()
    pltpu.sync_copy(acc.at[pl.ds(sid * SEGU_PER_TILE, SEGU_PER_TILE)],
                    partials_hbm.at[cid, pl.ds(sid * SEGU_PER_TILE, SEGU_PER_TILE)])


def _mlp_f32(x, w1, b1, w2, b2):
    h = lax.dot_general(x, w1, (((1,), (1,)), ((), ())),
                        preferred_element_type=jnp.float32)
    h = jnp.maximum(h + b1, 0.0)
    o = lax.dot_general(h, w2, (((1,), (1,)), ((), ())),
                        preferred_element_type=jnp.float32)
    return o + b2


def _mlp_bf16(x, w1, b1, w2, b2):
    h = lax.dot_general(x.astype(jnp.bfloat16), w1, (((1,), (1,)), ((), ())),
                        preferred_element_type=jnp.float32)
    h = jnp.maximum(h + b1, 0.0)
    o = lax.dot_general(h.astype(jnp.bfloat16), w2, (((1,), (1,)), ((), ())),
                        preferred_element_type=jnp.float32)
    return o + b2


def _rho_body(p_ref, w1_ref, b1_ref, w2_ref, b2_ref, o_ref):
    p = p_ref[0] + p_ref[1]
    o_ref[:] = _mlp_f32(p, w1_ref[:], b1_ref[:],
                        w2_ref[:], b2_ref[:]).astype(jnp.bfloat16)


def _phi_body(nl_ref, act_ref, batch_ref, x_ref, tab_ref,
              w1p_ref, b1p_ref, w2p_ref, b2p_ref, o_ref):
    phi = _mlp_bf16(x_ref[:], w1p_ref[:], b1p_ref[:], w2p_ref[:], b2p_ref[:])
    idxv = act_ref[0, 0, :] + batch_ref[0, 0, :] * nl_ref[0]
    m = (idxv.astype(jnp.int16)[:, None] ==
         lax.broadcasted_iota(jnp.int16, (1, SEGU), 1))
    onehot = lax.bitcast_convert_type(
        jnp.where(m, jnp.int16(0x3F80), jnp.int16(0)), jnp.bfloat16)
    rho = lax.dot_general(onehot, tab_ref[:], (((1,), (0,)), ((), ())),
                          preferred_element_type=jnp.float32)
    o_ref[:] = phi + rho


_BLK = 2000


def kernel(x, activation_idx, batch, num_layers,
           W1p, b1p, W2p, b2p, W1r, b1r, W2r, b2r):
    zeros = jnp.zeros((SEGU, D), jnp.float32)
    partials = _sc_scatter(x, activation_idx, batch, num_layers, zeros)

    rho_table = pl.pallas_call(
        _rho_body,
        out_shape=jax.ShapeDtypeStruct((SEGU, D), jnp.bfloat16),
    )(partials, W1r, b1r.reshape(1, D), W2r, b2r.reshape(1, D))

    act3 = activation_idx.reshape(N // _BLK, 1, _BLK)
    batch3 = batch.reshape(N // _BLK, 1, _BLK)
    ispec = pl.BlockSpec((1, 1, _BLK), lambda i: (i, 0, 0))
    wspec = pl.BlockSpec((D, D), lambda i: (0, 0))
    bspec = pl.BlockSpec((1, D), lambda i: (0, 0))
    out = pl.pallas_call(
        _phi_body,
        grid=(N // _BLK,),
        in_specs=[
            pl.BlockSpec(memory_space=pltpu.SMEM),
            ispec, ispec,
            pl.BlockSpec((_BLK, D), lambda i: (i, 0)),
            pl.BlockSpec((SEGU, D), lambda i: (0, 0)),
            wspec, bspec, wspec, bspec,
        ],
        out_specs=pl.BlockSpec((_BLK, D), lambda i: (i, 0)),
        out_shape=jax.ShapeDtypeStruct((N, D), jnp.float32),
    )(num_layers, act3, batch3, x, rho_table,
      W1p.astype(jnp.bfloat16), b1p.reshape(1, D),
      W2p.astype(jnp.bfloat16), b2p.reshape(1, D))
    return out


# speedup vs baseline: 6.1935x; 1.1908x over previous
import functools

import jax
import jax.numpy as jnp
from jax import lax
from jax.experimental import pallas as pl
from jax.experimental.pallas import tpu as pltpu
from jax.experimental.pallas import tpu_sc as plsc

N = 100000
D = 128
SEG = 1024
SEGU = 512
NC = 2
NS = 16
NW = NC * NS
S = 400
G = S // 80
NSUP = N // S
NFULL = NSUP % NW
KHI = -(-NSUP // NW)
SEGU_PER_TILE = SEGU // NS

_MESH = plsc.VectorSubcoreMesh(core_axis_name="c", subcore_axis_name="s")


def _compute_idx(abuf, bbuf, ibuf, nl0):
    for j in range(G):
        for t in range(5):
            av = abuf[pl.ds(j * 80 + t * 16, 16)]
            bv = bbuf[pl.ds(j * 80 + t * 16, 16)]
            ibuf[j, pl.ds(t * 16, 16)] = av + bv * nl0


@functools.partial(
    pl.kernel,
    mesh=_MESH,
    out_type=jax.ShapeDtypeStruct((NC, SEGU, D), jnp.float32),
    scratch_types=[
        pltpu.VMEM((S, D), jnp.float32),
        pltpu.VMEM((S, D), jnp.float32),
        pltpu.VMEM((S,), jnp.int32),
        pltpu.VMEM((S,), jnp.int32),
        pltpu.VMEM((S,), jnp.int32),
        pltpu.VMEM((S,), jnp.int32),
        pltpu.VMEM((G, 80), jnp.int32),
        pltpu.VMEM((G, 80), jnp.int32),
        pltpu.VMEM((16,), jnp.int32),
        pltpu.VMEM((SEGU_PER_TILE, D), jnp.float32),
        pltpu.VMEM_SHARED((SEGU, D), jnp.float32),
        pltpu.SemaphoreType.DMA,
        pltpu.SemaphoreType.DMA,
        pltpu.SemaphoreType.DMA,
    ],
)
def _sc_scatter(x_hbm, act_hbm, batch_hbm, nl_hbm,
                partials_hbm,
                xbuf0, xbuf1, abuf0, abuf1, bbuf0, bbuf1, ibuf0, ibuf1,
                nlbuf, zbuf, acc, isem0, isem1, ssem):
    cid = lax.axis_index("c")
    sid = lax.axis_index("s")
    w = sid * NC + cid
    nsup = jnp.where(w < NFULL, KHI, KHI - 1)
    xbufs, abufs, bbufs, ibufs = (xbuf0, xbuf1), (abuf0, abuf1), (bbuf0, bbuf1), (ibuf0, ibuf1)
    isems = (isem0, isem1)

    def stage(s_id, b):
        base = (w + NW * s_id) * S
        pltpu.async_copy(x_hbm.at[pl.ds(base, S)], xbufs[b], isems[b])
        pltpu.async_copy(act_hbm.at[pl.ds(base, S)], abufs[b], isems[b])
        pltpu.async_copy(batch_hbm.at[pl.ds(base, S)], bbufs[b], isems[b])

    zv = jnp.zeros((16,), jnp.float32)
    for r in range(SEGU_PER_TILE):
        for c in range(D // 16):
            zbuf[r, pl.ds(c * 16, 16)] = zv
    pltpu.sync_copy(zbuf, acc.at[pl.ds(sid * SEGU_PER_TILE, SEGU_PER_TILE)])
    pltpu.sync_copy(nl_hbm.at[pl.ds(0, 16)], nlbuf)
    for b in range(2):
        stage(b, b)
    nl0 = nlbuf[pl.ds(0, 16)][0]
    plsc.subcore_barrier()

    def round_body(g, carry):
        for b in range(2):
            s_id = 2 * g + b

            @pl.when(s_id < nsup)
            def _():
                pltpu.make_async_copy(x_hbm.at[pl.ds(0, S)], xbufs[b], isems[b]).wait()
                pltpu.make_async_copy(act_hbm.at[pl.ds(0, S)], abufs[b], isems[b]).wait()
                pltpu.make_async_copy(batch_hbm.at[pl.ds(0, S)], bbufs[b], isems[b]).wait()

                _compute_idx(abufs[b], bbufs[b], ibufs[b], nl0)
                for j in range(G):
                    pltpu.async_copy(xbufs[b].at[pl.ds(j * 80, 80)],
                                     acc.at[ibufs[b].at[j]], ssem, add=True)
                for j in range(G):
                    pltpu.make_async_copy(xbufs[b].at[pl.ds(j * 80, 80)],
                                          acc.at[ibufs[b].at[j]], ssem).wait()

                @pl.when(s_id + 2 < nsup)
                def _():
                    stage(s_id + 2, b)

        return carry

    lax.fori_loop(0, KHI // 2, round_body, 0)
    plsc.subcore_barrier()
    pltpu.sync_copy(acc.at[pl.ds(sid * SEGU_PER_TILE, SEGU_PER_TILE)],
                    partials_hbm.at[cid, pl.ds(sid * SEGU_PER_TILE, SEGU_PER_TILE)])


def _mlp_f32(x, w1, b1, w2, b2):
    h = lax.dot_general(x, w1, (((1,), (1,)), ((), ())),
                        preferred_element_type=jnp.float32)
    h = jnp.maximum(h + b1, 0.0)
    o = lax.dot_general(h, w2, (((1,), (1,)), ((), ())),
                        preferred_element_type=jnp.float32)
    return o + b2


def _mlp_bf16(x, w1, b1, w2, b2):
    h = lax.dot_general(x.astype(jnp.bfloat16), w1.astype(jnp.bfloat16),
                        (((1,), (1,)), ((), ())),
                        preferred_element_type=jnp.float32)
    h = jnp.maximum(h + b1, 0.0)
    o = lax.dot_general(h.astype(jnp.bfloat16), w2.astype(jnp.bfloat16),
                        (((1,), (1,)), ((), ())),
                        preferred_element_type=jnp.float32)
    return o + b2


def _rho_body(p_ref, w1_ref, b1_ref, w2_ref, b2_ref, o_ref):
    p = p_ref[0] + p_ref[1]
    o_ref[:] = _mlp_f32(p, w1_ref[:], b1_ref[:],
                        w2_ref[:], b2_ref[:]).astype(jnp.bfloat16)


def _phi_body(nl_ref, act_ref, batch_ref, x_ref, tab_ref,
              w1p_ref, b1p_ref, w2p_ref, b2p_ref, o_ref):
    phi = _mlp_bf16(x_ref[:], w1p_ref[:], b1p_ref[:], w2p_ref[:], b2p_ref[:])
    idxv = act_ref[0, 0, :] + batch_ref[0, 0, :] * nl_ref[0]
    m = (idxv.astype(jnp.int16)[:, None] ==
         lax.broadcasted_iota(jnp.int16, (1, SEGU), 1))
    onehot = lax.bitcast_convert_type(
        jnp.where(m, jnp.int16(0x3F80), jnp.int16(0)), jnp.bfloat16)
    rho = lax.dot_general(onehot, tab_ref[:], (((1,), (0,)), ((), ())),
                          preferred_element_type=jnp.float32)
    o_ref[:] = phi + rho


_BLK = 4000


def kernel(x, activation_idx, batch, num_layers,
           W1p, b1p, W2p, b2p, W1r, b1r, W2r, b2r):
    partials = _sc_scatter(x, activation_idx, batch, num_layers)

    rho_table = pl.pallas_call(
        _rho_body,
        out_shape=jax.ShapeDtypeStruct((SEGU, D), jnp.bfloat16),
    )(partials, W1r, b1r.reshape(1, D), W2r, b2r.reshape(1, D))

    act3 = activation_idx.reshape(N // _BLK, 1, _BLK)
    batch3 = batch.reshape(N // _BLK, 1, _BLK)
    ispec = pl.BlockSpec((1, 1, _BLK), lambda i: (i, 0, 0))
    wspec = pl.BlockSpec((D, D), lambda i: (0, 0))
    bspec = pl.BlockSpec((1, D), lambda i: (0, 0))
    out = pl.pallas_call(
        _phi_body,
        grid=(N // _BLK,),
        in_specs=[
            pl.BlockSpec(memory_space=pltpu.SMEM),
            ispec, ispec,
            pl.BlockSpec((_BLK, D), lambda i: (i, 0)),
            pl.BlockSpec((SEGU, D), lambda i: (0, 0)),
            wspec, bspec, wspec, bspec,
        ],
        out_specs=pl.BlockSpec((_BLK, D), lambda i: (i, 0)),
        out_shape=jax.ShapeDtypeStruct((N, D), jnp.float32),
    )(num_layers, act3, batch3, x, rho_table,
      W1p, b1p.reshape(1, D), W2p, b2p.reshape(1, D))
    return out
